# Initial kernel scaffold; baseline (speedup 1.0000x reference)
import jax
import jax.numpy as jnp
from jax.experimental import pallas as pl


def kernel(q, Q, omega, e_tilde, e, g, G):
    raise NotImplementedError("write your pallas kernel here")


# speedup vs baseline: 1.2567x; 1.2567x over previous
import jax
import jax.numpy as jnp
from jax.experimental import pallas as pl
from jax.experimental.pallas import tpu as pltpu

NA, NK, NM, ND, NG_ = 3, 8, 32, 32, 12
NAK = NA * NK
BLOCK = 128


def _unfold_kernel(qT_ref, QT_ref, gG_ref, om_r_ref, om_c_ref, et_ref, e_ref,
                   out_ref, P_scr):
    ak = pl.program_id(1)

    maskT = None
    for c in range(3):
        unf = qT_ref[c : c + 1, :] + gG_ref[:, c : c + 1]
        diff = QT_ref[c : c + 1, :] - unf
        cond = jnp.abs(diff) <= 1e-5 + 1e-5 * jnp.abs(unf)
        maskT = cond if maskT is None else jnp.logical_and(maskT, cond)
    wmask = maskT.astype(jnp.float32)

    e_ak = e_ref[0, 0]
    es = jnp.sum(e_ak * wmask[None, :, :], axis=1)
    t_ak = et_ref[0, 0]
    dots = jnp.sum(t_ak * es.T[None, :, :], axis=-1)
    sq = dots * dots

    @pl.when(ak == 0)
    def _init():
        P_scr[...] = sq

    @pl.when(ak != 0)
    def _acc():
        P_scr[...] = P_scr[...] + sq

    @pl.when(ak == NAK - 1)
    def _finish():
        eq = (om_r_ref[...] == om_c_ref[...]).astype(jnp.float32)
        out_ref[...] = jnp.dot(
            P_scr[...].T, eq, preferred_element_type=jnp.float32
        ) * (4.0 / 12.0)


@jax.jit
def kernel(q, Q, omega, e_tilde, e, g, G):
    nq = q.shape[0]
    qT = q.T
    QT = Q.T
    gG = g - G[None, :]
    om_r = omega.reshape(NM, 1)
    om_c = omega.reshape(1, NM)

    grid = (nq // BLOCK, NAK)
    out = pl.pallas_call(
        _unfold_kernel,
        grid=grid,
        in_specs=[
            pl.BlockSpec((3, BLOCK), lambda b, ak: (0, b)),
            pl.BlockSpec((3, BLOCK), lambda b, ak: (0, b)),
            pl.BlockSpec((NG_, 3), lambda b, ak: (0, 0)),
            pl.BlockSpec((NM, 1), lambda b, ak: (0, 0)),
            pl.BlockSpec((1, NM), lambda b, ak: (0, 0)),
            pl.BlockSpec((1, 1, NM, BLOCK, ND),
                         lambda b, ak: (ak // NK, ak % NK, 0, b, 0)),
            pl.BlockSpec((1, 1, ND, NG_, BLOCK),
                         lambda b, ak: (ak // NK, ak % NK, 0, 0, b)),
        ],
        out_specs=pl.BlockSpec((BLOCK, NM), lambda b, ak: (b, 0)),
        out_shape=jax.ShapeDtypeStruct((nq, NM), jnp.float32),
        scratch_shapes=[pltpu.VMEM((NM, BLOCK), jnp.float32)],
    )(qT, QT, gG, om_r, om_c, e_tilde, e)
    return out
